# baseline (speedup 1.0000x reference)
import functools

import jax
import jax.numpy as jnp
from jax import lax
from jax.experimental import pallas as pl
from jax.experimental.pallas import tpu as pltpu
from jax.experimental.pallas import tpu_sc as plsc

_NC, _NS, _L = 2, 16, 16
_NW = _NC * _NS


def _lane_shuffle(x, perm):
    return lax.gather(
        x, perm[:, None],
        lax.GatherDimensionNumbers(
            offset_dims=(), collapsed_slice_dims=(0,), start_index_map=(0,)),
        slice_sizes=(1,),
        mode=lax.GatherScatterMode.PROMISE_IN_BOUNDS)


def _tc_argmax_kernel(x_ref, out_ref, *, chunk):
    rows, v = x_ref.shape
    run_max = jnp.full((rows,), -jnp.inf, dtype=jnp.float32)
    run_idx = jnp.zeros((rows,), dtype=jnp.int32)
    off = 0
    while off < v:
        w = min(chunk, v - off)
        xc = x_ref[:, off:off + w]
        cmax = jnp.max(xc, axis=1)
        cidx = jnp.argmax(xc, axis=1).astype(jnp.int32) + off
        upd = cmax > run_max
        run_max = jnp.where(upd, cmax, run_max)
        run_idx = jnp.where(upd, cidx, run_idx)
        off += w
    out_ref[...] = run_idx.reshape(out_ref.shape)


def _tc_argmax(logits, rows_per_block=32, chunk=12800):
    r, v = logits.shape
    return pl.pallas_call(
        functools.partial(_tc_argmax_kernel, chunk=chunk),
        grid=(r // rows_per_block,),
        in_specs=[pl.BlockSpec((rows_per_block, v), lambda i: (i, 0))],
        out_specs=pl.BlockSpec((rows_per_block, 1), lambda i: (i, 0)),
        out_shape=jax.ShapeDtypeStruct((r, 1), jnp.int32),
        compiler_params=pltpu.CompilerParams(
            dimension_semantics=("parallel",)),
    )(logits)[:, 0]


def _sc_argmax(logits_flat, nrows, v, ch=20000, unroll=10):
    rpw = nrows // _NW
    nch = v // ch
    vecs = ch // _L
    steps = vecs // unroll
    stride = unroll * _L
    mesh = plsc.VectorSubcoreMesh(core_axis_name="c", subcore_axis_name="s")

    @functools.partial(
        pl.kernel,
        out_type=jax.ShapeDtypeStruct((_NW, _L), jnp.int32),
        mesh=mesh,
        scratch_types=[
            pltpu.VMEM((ch,), jnp.float32),
            pltpu.VMEM((ch,), jnp.float32),
            pltpu.VMEM((_L,), jnp.int32),
            pltpu.SemaphoreType.DMA,
            pltpu.SemaphoreType.DMA,
        ],
    )
    def k(x_hbm, out_hbm, vbuf0, vbuf1, obuf, sem0, sem1):
        vbufs = (vbuf0, vbuf1)
        wid = lax.axis_index("s") * _NC + lax.axis_index("c")
        sems = (sem0, sem1)
        base_row = wid * rpw
        iota = lax.iota(jnp.int32, _L)

        def chunk_copy(row, c, b):
            off = pl.multiple_of((base_row + row) * v + c * ch, 8)
            return pltpu.make_async_copy(
                x_hbm.at[pl.ds(off, ch)], vbufs[b], sems[b])

        chunk_copy(0, 0, 0).start()
        out_vec = jnp.zeros((_L,), jnp.int32)
        for r in range(rpw):
            accs_v = [jnp.full((_L,), -jnp.inf, jnp.float32)
                      for _ in range(unroll)]
            accs_t = [jnp.zeros((_L,), jnp.int32) for _ in range(unroll)]
            tvec = jnp.zeros((_L,), jnp.int32)
            for c in range(nch):
                b = (r * nch + c) % 2
                chunk_copy(r, c, b).wait()
                if c + 1 < nch:
                    chunk_copy(r, c + 1, (b + 1) % 2).start()
                elif r + 1 < rpw:
                    chunk_copy(r + 1, 0, (b + 1) % 2).start()

                def body(i, carry):
                    av = list(carry[:unroll])
                    at = list(carry[unroll:2 * unroll])
                    tv = carry[2 * unroll]
                    base = i * stride
                    for j in range(unroll):
                        x = vbufs[b][pl.ds(base + j * _L, _L)]
                        m = x > av[j]
                        av[j] = jnp.where(m, x, av[j])
                        at[j] = jnp.where(m, tv, at[j])
                    return (*av, *at, tv + 1)

                carry = lax.fori_loop(
                    0, steps, body, (*accs_v, *accs_t, tvec))
                accs_v = list(carry[:unroll])
                accs_t = list(carry[unroll:2 * unroll])
                tvec = carry[2 * unroll]

            bv = accs_v[0]
            bl = accs_t[0] * stride + iota
            for j in range(1, unroll):
                lv = accs_v[j]
                ll = accs_t[j] * stride + (j * _L) + iota
                better = (lv > bv) | ((lv == bv) & (ll < bl))
                bv = jnp.where(better, lv, bv)
                bl = jnp.where(better, ll, bl)
            for sh in (8, 4, 2, 1):
                perm = iota ^ sh
                gv = _lane_shuffle(bv, perm)
                gl = _lane_shuffle(bl, perm)
                better = (gv > bv) | ((gv == bv) & (gl < bl))
                bv = jnp.where(better, gv, bv)
                bl = jnp.where(better, gl, bl)
            out_vec = jnp.where(iota == r, bl, out_vec)

        obuf[...] = out_vec
        pltpu.sync_copy(obuf, out_hbm.at[wid])

    out = k(logits_flat)
    return out[:, :rpw].reshape(nrows)


def _accept_kernel(ids_ref, spec_ref, sampled_ref, len_ref):
    ids = ids_ref[...]
    spec = spec_ref[...]
    B, K1 = ids.shape
    K = K1 - 1
    prod = jnp.ones((B,), dtype=jnp.int32)
    total = jnp.zeros((B,), dtype=jnp.int32)
    for k in range(K):
        m = (ids[:, k] == spec[:, k]).astype(jnp.int32)
        prod = prod * m
        total = total + prod
    lengths = total + 1
    pos = jax.lax.broadcasted_iota(jnp.int32, (B, K1), 1)
    sampled_ref[...] = jnp.where(pos < lengths[:, None], ids, -1)
    len_ref[...] = lengths.reshape(B, 1)


_TC_ROWS = 0


def kernel(logits, spec_token_ids):
    B, K = spec_token_ids.shape
    R, V = logits.shape
    parts = []
    if _TC_ROWS > 0:
        parts.append(_tc_argmax(logits[:_TC_ROWS]))
    if _TC_ROWS < R:
        sc_rows = R - _TC_ROWS
        flat = logits[_TC_ROWS:].reshape(sc_rows * V)
        parts.append(_sc_argmax(flat, sc_rows, V))
    out_ids = parts[0] if len(parts) == 1 else jnp.concatenate(parts)
    out_ids = out_ids.reshape(B, K + 1)
    sampled, lengths = pl.pallas_call(
        _accept_kernel,
        in_specs=[
            pl.BlockSpec((B, K + 1), lambda: (0, 0)),
            pl.BlockSpec((B, K), lambda: (0, 0)),
        ],
        out_specs=[
            pl.BlockSpec((B, K + 1), lambda: (0, 0)),
            pl.BlockSpec((B, 1), lambda: (0, 0)),
        ],
        out_shape=[
            jax.ShapeDtypeStruct((B, K + 1), jnp.int32),
            jax.ShapeDtypeStruct((B, 1), jnp.int32),
        ],
    )(out_ids, spec_token_ids)
    return sampled, lengths.reshape(B)


# speedup vs baseline: 3.7417x; 3.7417x over previous
import functools

import jax
import jax.numpy as jnp
from jax import lax
from jax.experimental import pallas as pl
from jax.experimental.pallas import tpu as pltpu
from jax.experimental.pallas import tpu_sc as plsc

_NC, _NS, _L = 2, 16, 16
_NW = _NC * _NS


def _lane_shuffle(x, perm):
    return lax.gather(
        x, perm[:, None],
        lax.GatherDimensionNumbers(
            offset_dims=(), collapsed_slice_dims=(0,), start_index_map=(0,)),
        slice_sizes=(1,),
        mode=lax.GatherScatterMode.PROMISE_IN_BOUNDS)


def _tc_argmax_kernel(x_ref, out_ref, *, chunk):
    rows, v = x_ref.shape
    run_max = jnp.full((rows,), -jnp.inf, dtype=jnp.float32)
    run_idx = jnp.zeros((rows,), dtype=jnp.int32)
    off = 0
    while off < v:
        w = min(chunk, v - off)
        xc = x_ref[:, off:off + w]
        cmax = jnp.max(xc, axis=1)
        cidx = jnp.argmax(xc, axis=1).astype(jnp.int32) + off
        upd = cmax > run_max
        run_max = jnp.where(upd, cmax, run_max)
        run_idx = jnp.where(upd, cidx, run_idx)
        off += w
    out_ref[...] = run_idx.reshape(out_ref.shape)


def _tc_argmax(logits, nrows, rows_per_block=32, chunk=12800):
    _, v = logits.shape
    return pl.pallas_call(
        functools.partial(_tc_argmax_kernel, chunk=chunk),
        grid=(nrows // rows_per_block,),
        in_specs=[pl.BlockSpec((rows_per_block, v), lambda i: (i, 0))],
        out_specs=pl.BlockSpec((rows_per_block, 1), lambda i: (i, 0)),
        out_shape=jax.ShapeDtypeStruct((nrows, 1), jnp.int32),
        compiler_params=pltpu.CompilerParams(
            dimension_semantics=("parallel",)),
    )(logits)[:, 0]


_W = 7168
_NCHK = 7
_HBASE = 49792
_TAIL0 = _NCHK * _W + _HBASE - _W
_U = 8
_STEPS = (_W // _L) // _U
_STRIDE = _U * _L


def _sc_argmax(logits, row0):
    mesh = plsc.VectorSubcoreMesh(core_axis_name="c", subcore_axis_name="s")

    @functools.partial(
        pl.kernel,
        out_type=(jax.ShapeDtypeStruct((_NW, _L), jnp.float32),
                  jax.ShapeDtypeStruct((_NW, _L), jnp.int32)),
        mesh=mesh,
        scratch_types=[
            pltpu.VMEM((8, _W), jnp.float32),
            pltpu.VMEM((8, _W), jnp.float32),
            pltpu.VMEM((_L,), jnp.float32),
            pltpu.VMEM((_L,), jnp.int32),
            pltpu.SemaphoreType.DMA,
            pltpu.SemaphoreType.DMA,
        ],
    )
    def k(x_hbm, outv_hbm, outl_hbm, vbuf0, vbuf1, obv, obl, sem0, sem1):
        vbufs = (vbuf0, vbuf1)
        sems = (sem0, sem1)
        hi = lax.axis_index("c")
        gi = lax.axis_index("s")
        wid = hi * _NS + gi
        iota = lax.iota(jnp.int32, _L)
        grow = pl.multiple_of(row0 + gi * 8, 8)
        cbase = pl.multiple_of(hi * _HBASE, 128)

        def chunk_copy(c, b):
            off = pl.multiple_of(cbase + c * _W, 128)
            return pltpu.make_async_copy(
                x_hbm.at[pl.ds(grow, 8), pl.ds(off, _W)],
                vbufs[b], sems[b])

        chunk_copy(0, 0).start()
        vv = jnp.full((_L,), -jnp.inf, jnp.float32)
        vl = jnp.zeros((_L,), jnp.int32)
        for c in range(_NCHK):
            b = c % 2
            chunk_copy(c, b).wait()
            if c + 1 < _NCHK:
                chunk_copy(c + 1, (b + 1) % 2).start()
            off = cbase + c * _W

            def jbody(j, carry):
                vv, vl = carry
                accs_v = [jnp.full((_L,), -jnp.inf, jnp.float32)
                          for _ in range(_U)]
                accs_l = [jnp.zeros((_L,), jnp.int32) for _ in range(_U)]
                linb = off + iota

                def body(i, carry):
                    av = list(carry[:_U])
                    al = list(carry[_U:2 * _U])
                    lb = carry[2 * _U]
                    base = i * _STRIDE
                    for u in range(_U):
                        x = vbufs[b][j, pl.ds(base + u * _L, _L)]
                        m = x > av[u]
                        av[u] = jnp.where(m, x, av[u])
                        al[u] = jnp.where(m, lb, al[u])
                    return (*av, *al, lb + _STRIDE)

                carry2 = lax.fori_loop(
                    0, _STEPS, body, (*accs_v, *accs_l, linb))
                accs_v = list(carry2[:_U])
                accs_l = list(carry2[_U:2 * _U])
                bv = accs_v[0]
                bl = accs_l[0]
                for u in range(1, _U):
                    lv = accs_v[u]
                    ll = accs_l[u] + u * _L
                    better = (lv > bv) | ((lv == bv) & (ll < bl))
                    bv = jnp.where(better, lv, bv)
                    bl = jnp.where(better, ll, bl)
                for sh in (8, 4, 2, 1):
                    perm = iota ^ sh
                    gv = _lane_shuffle(bv, perm)
                    gl = _lane_shuffle(bl, perm)
                    better = (gv > bv) | ((gv == bv) & (gl < bl))
                    bv = jnp.where(better, gv, bv)
                    bl = jnp.where(better, gl, bl)
                upd = (iota == j) & ((bv > vv) | ((bv == vv) & (bl < vl)))
                vv = jnp.where(upd, bv, vv)
                vl = jnp.where(upd, bl, vl)
                return (vv, vl)

            vv, vl = lax.fori_loop(0, 8, jbody, (vv, vl))

        obv[...] = vv
        obl[...] = vl
        pltpu.sync_copy(obv, outv_hbm.at[wid])
        pltpu.sync_copy(obl, outl_hbm.at[wid])

    return k(logits)


def _merge_kernel(tail_ref, v_ref, l_ref, out_ref):
    v = v_ref[...]
    l = l_ref[...]
    v0, v1 = v[0:_NS], v[_NS:]
    l0, l1 = l[0:_NS], l[_NS:]
    take1 = (v1 > v0) | ((v1 == v0) & (l1 < l0))
    bv = jnp.where(take1, v1, v0)
    bl = jnp.where(take1, l1, l0)
    t = tail_ref[...]
    tv = jnp.max(t, axis=2)
    tl = jnp.argmax(t, axis=2).astype(jnp.int32) + _TAIL0
    pad_v = jnp.full((_NS, _L - 8), -jnp.inf, jnp.float32)
    pad_l = jnp.zeros((_NS, _L - 8), jnp.int32)
    tv16 = jnp.concatenate([tv, pad_v], axis=1)
    tl16 = jnp.concatenate([tl, pad_l], axis=1)
    taket = tv16 > bv
    out_ref[...] = jnp.where(taket, tl16, bl)


def _accept_kernel(ids_ref, spec_ref, sampled_ref, len_ref):
    ids = ids_ref[...]
    spec = spec_ref[...]
    B, K1 = ids.shape
    K = K1 - 1
    prod = jnp.ones((B,), dtype=jnp.int32)
    total = jnp.zeros((B,), dtype=jnp.int32)
    for k in range(K):
        m = (ids[:, k] == spec[:, k]).astype(jnp.int32)
        prod = prod * m
        total = total + prod
    lengths = total + 1
    pos = jax.lax.broadcasted_iota(jnp.int32, (B, K1), 1)
    sampled_ref[...] = jnp.where(pos < lengths[:, None], ids, -1)
    len_ref[...] = lengths.reshape(B, 1)


_TC_ROWS = 192


def kernel(logits, spec_token_ids):
    B, K = spec_token_ids.shape
    R, V = logits.shape
    tc_ids = _tc_argmax(logits, _TC_ROWS)
    scv, scl = _sc_argmax(logits, _TC_ROWS)
    tail = logits[_TC_ROWS:, _TAIL0:].reshape(_NS, 8, V - _TAIL0)
    sc_ids = pl.pallas_call(
        _merge_kernel,
        in_specs=[
            pl.BlockSpec((_NS, 8, V - _TAIL0), lambda: (0, 0, 0)),
            pl.BlockSpec((_NW, _L), lambda: (0, 0)),
            pl.BlockSpec((_NW, _L), lambda: (0, 0)),
        ],
        out_specs=pl.BlockSpec((_NS, _L), lambda: (0, 0)),
        out_shape=jax.ShapeDtypeStruct((_NS, _L), jnp.int32),
    )(tail, scv, scl)
    out_ids = jnp.concatenate([tc_ids, sc_ids[:, :8].reshape(R - _TC_ROWS)])
    out_ids = out_ids.reshape(B, K + 1)
    sampled, lengths = pl.pallas_call(
        _accept_kernel,
        in_specs=[
            pl.BlockSpec((B, K + 1), lambda: (0, 0)),
            pl.BlockSpec((B, K), lambda: (0, 0)),
        ],
        out_specs=[
            pl.BlockSpec((B, K + 1), lambda: (0, 0)),
            pl.BlockSpec((B, 1), lambda: (0, 0)),
        ],
        out_shape=[
            jax.ShapeDtypeStruct((B, K + 1), jnp.int32),
            jax.ShapeDtypeStruct((B, 1), jnp.int32),
        ],
    )(out_ids, spec_token_ids)
    return sampled, lengths.reshape(B)


# speedup vs baseline: 5.4987x; 1.4696x over previous
import functools

import jax
import jax.numpy as jnp
from jax import lax
from jax.experimental import pallas as pl
from jax.experimental.pallas import tpu as pltpu
from jax.experimental.pallas import tpu_sc as plsc

_NC, _NS, _L = 2, 16, 16
_NW = _NC * _NS


def _lane_shuffle(x, perm):
    return lax.gather(
        x, perm[:, None],
        lax.GatherDimensionNumbers(
            offset_dims=(), collapsed_slice_dims=(0,), start_index_map=(0,)),
        slice_sizes=(1,),
        mode=lax.GatherScatterMode.PROMISE_IN_BOUNDS)


def _tc_argmax_kernel(x_ref, out_ref, *, chunk):
    rows, v = x_ref.shape
    run_max = jnp.full((rows,), -jnp.inf, dtype=jnp.float32)
    run_idx = jnp.zeros((rows,), dtype=jnp.int32)
    off = 0
    while off < v:
        w = min(chunk, v - off)
        xc = x_ref[:, off:off + w]
        cmax = jnp.max(xc, axis=1)
        cidx = jnp.argmax(xc, axis=1).astype(jnp.int32) + off
        upd = cmax > run_max
        run_max = jnp.where(upd, cmax, run_max)
        run_idx = jnp.where(upd, cidx, run_idx)
        off += w
    out_ref[...] = run_idx.reshape(out_ref.shape)


def _tc_ring_kernel(x_hbm, out_ref, *bufs_sems, nbuf, rb, chunk):
    nrows, v = out_ref.shape[0], x_hbm.shape[1]
    bufs = bufs_sems[:nbuf]
    sems = bufs_sems[nbuf:]
    n = nrows // rb

    def copy(i):
        return pltpu.make_async_copy(
            x_hbm.at[pl.ds(i * rb, rb), :], bufs[i % nbuf], sems[i % nbuf])

    for i in range(min(nbuf, n)):
        copy(i).start()
    for i in range(n):
        copy(i).wait()
        buf = bufs[i % nbuf]
        run_max = jnp.full((rb,), -jnp.inf, dtype=jnp.float32)
        run_idx = jnp.zeros((rb,), dtype=jnp.int32)
        off = 0
        while off < v:
            w = min(chunk, v - off)
            xc = buf[:, off:off + w]
            cmax = jnp.max(xc, axis=1)
            cidx = jnp.argmax(xc, axis=1).astype(jnp.int32) + off
            upd = cmax > run_max
            run_max = jnp.where(upd, cmax, run_max)
            run_idx = jnp.where(upd, cidx, run_idx)
            off += w
        out_ref[pl.ds(i * rb, rb), :] = run_idx.reshape(rb, 1)
        if i + nbuf < n:
            copy(i + nbuf).start()


def _tc_argmax_ring(logits, nrows, nbuf=4, rb=16, chunk=12800):
    _, v = logits.shape
    return pl.pallas_call(
        functools.partial(_tc_ring_kernel, nbuf=nbuf, rb=rb, chunk=chunk),
        in_specs=[pl.BlockSpec(memory_space=pl.ANY)],
        out_specs=pl.BlockSpec((nrows, 1), lambda: (0, 0)),
        out_shape=jax.ShapeDtypeStruct((nrows, 1), jnp.int32),
        scratch_shapes=(
            [pltpu.VMEM((rb, v), jnp.float32)] * nbuf
            + [pltpu.SemaphoreType.DMA] * nbuf),
    )(logits)[:, 0]


def _tc_argmax(logits, nrows, rows_per_block=32, chunk=12800):
    _, v = logits.shape
    return pl.pallas_call(
        functools.partial(_tc_argmax_kernel, chunk=chunk),
        grid=(nrows // rows_per_block,),
        in_specs=[pl.BlockSpec((rows_per_block, v), lambda i: (i, 0))],
        out_specs=pl.BlockSpec((rows_per_block, 1), lambda i: (i, 0)),
        out_shape=jax.ShapeDtypeStruct((nrows, 1), jnp.int32),
        compiler_params=pltpu.CompilerParams(
            dimension_semantics=("parallel",)),
    )(logits)[:, 0]


_W = 7168
_NCHK = 7
_HBASE = 49792
_TAIL0 = _NCHK * _W + _HBASE - _W
_U = 8
_STEPS = (_W // _L) // _U
_STRIDE = _U * _L


def _sc_argmax(logits, row0):
    mesh = plsc.VectorSubcoreMesh(core_axis_name="c", subcore_axis_name="s")

    @functools.partial(
        pl.kernel,
        out_type=(jax.ShapeDtypeStruct((_NW, _L), jnp.float32),
                  jax.ShapeDtypeStruct((_NW, _L), jnp.int32)),
        mesh=mesh,
        scratch_types=[
            pltpu.VMEM((8, _W), jnp.float32),
            pltpu.VMEM((8, _W), jnp.float32),
            pltpu.VMEM((_L,), jnp.float32),
            pltpu.VMEM((_L,), jnp.int32),
            pltpu.SemaphoreType.DMA,
            pltpu.SemaphoreType.DMA,
        ],
    )
    def k(x_hbm, outv_hbm, outl_hbm, vbuf0, vbuf1, obv, obl, sem0, sem1):
        vbufs = (vbuf0, vbuf1)
        sems = (sem0, sem1)
        hi = lax.axis_index("c")
        gi = lax.axis_index("s")
        wid = hi * _NS + gi
        iota = lax.iota(jnp.int32, _L)
        grow = pl.multiple_of(row0 + gi * 8, 8)
        cbase = pl.multiple_of(hi * _HBASE, 128)

        def chunk_copy(c, b):
            off = pl.multiple_of(cbase + c * _W, 128)
            return pltpu.make_async_copy(
                x_hbm.at[pl.ds(grow, 8), pl.ds(off, _W)],
                vbufs[b], sems[b])

        chunk_copy(0, 0).start()
        vv = jnp.full((_L,), -jnp.inf, jnp.float32)
        vl = jnp.zeros((_L,), jnp.int32)
        for c in range(_NCHK):
            b = c % 2
            chunk_copy(c, b).wait()
            if c + 1 < _NCHK:
                chunk_copy(c + 1, (b + 1) % 2).start()
            off = cbase + c * _W

            def jbody(j, carry):
                vv, vl = carry
                accs_v = [jnp.full((_L,), -jnp.inf, jnp.float32)
                          for _ in range(_U)]
                accs_l = [jnp.zeros((_L,), jnp.int32) for _ in range(_U)]
                linb = off + iota

                def body(i, carry):
                    av = list(carry[:_U])
                    al = list(carry[_U:2 * _U])
                    lb = carry[2 * _U]
                    base = i * _STRIDE
                    for u in range(_U):
                        x = vbufs[b][j, pl.ds(base + u * _L, _L)]
                        m = x > av[u]
                        av[u] = jnp.where(m, x, av[u])
                        al[u] = jnp.where(m, lb, al[u])
                    return (*av, *al, lb + _STRIDE)

                carry2 = lax.fori_loop(
                    0, _STEPS, body, (*accs_v, *accs_l, linb))
                accs_v = list(carry2[:_U])
                accs_l = list(carry2[_U:2 * _U])
                bv = accs_v[0]
                bl = accs_l[0]
                for u in range(1, _U):
                    lv = accs_v[u]
                    ll = accs_l[u] + u * _L
                    better = (lv > bv) | ((lv == bv) & (ll < bl))
                    bv = jnp.where(better, lv, bv)
                    bl = jnp.where(better, ll, bl)
                for sh in (8, 4, 2, 1):
                    perm = iota ^ sh
                    gv = _lane_shuffle(bv, perm)
                    gl = _lane_shuffle(bl, perm)
                    better = (gv > bv) | ((gv == bv) & (gl < bl))
                    bv = jnp.where(better, gv, bv)
                    bl = jnp.where(better, gl, bl)
                upd = (iota == j) & ((bv > vv) | ((bv == vv) & (bl < vl)))
                vv = jnp.where(upd, bv, vv)
                vl = jnp.where(upd, bl, vl)
                return (vv, vl)

            vv, vl = lax.fori_loop(0, 8, jbody, (vv, vl))

        obv[...] = vv
        obl[...] = vl
        pltpu.sync_copy(obv, outv_hbm.at[wid])
        pltpu.sync_copy(obl, outl_hbm.at[wid])

    return k(logits)


def _merge_kernel(tail_ref, v_ref, l_ref, out_ref):
    v = v_ref[...]
    l = l_ref[...]
    v0, v1 = v[0:_NS], v[_NS:]
    l0, l1 = l[0:_NS], l[_NS:]
    take1 = (v1 > v0) | ((v1 == v0) & (l1 < l0))
    bv = jnp.where(take1, v1, v0)
    bl = jnp.where(take1, l1, l0)
    t = tail_ref[...]
    tv = jnp.max(t, axis=2)
    tl = jnp.argmax(t, axis=2).astype(jnp.int32) + _TAIL0
    pad_v = jnp.full((_NS, _L - 8), -jnp.inf, jnp.float32)
    pad_l = jnp.zeros((_NS, _L - 8), jnp.int32)
    tv16 = jnp.concatenate([tv, pad_v], axis=1)
    tl16 = jnp.concatenate([tl, pad_l], axis=1)
    taket = tv16 > bv
    out_ref[...] = jnp.where(taket, tl16, bl)


def _accept_kernel(ids_ref, spec_ref, sampled_ref, len_ref):
    ids = ids_ref[...]
    spec = spec_ref[...]
    B, K1 = ids.shape
    K = K1 - 1
    prod = jnp.ones((B,), dtype=jnp.int32)
    total = jnp.zeros((B,), dtype=jnp.int32)
    for k in range(K):
        m = (ids[:, k] == spec[:, k]).astype(jnp.int32)
        prod = prod * m
        total = total + prod
    lengths = total + 1
    pos = jax.lax.broadcasted_iota(jnp.int32, (B, K1), 1)
    sampled_ref[...] = jnp.where(pos < lengths[:, None], ids, -1)
    len_ref[...] = lengths.reshape(B, 1)


_TC_ROWS = 320


def kernel(logits, spec_token_ids):
    B, K = spec_token_ids.shape
    R, V = logits.shape
    if _TC_ROWS == R:
        out_ids = _tc_argmax_ring(logits, R).reshape(B, K + 1)
        return _accept(out_ids, spec_token_ids, B, K)
    tc_ids = _tc_argmax(logits, _TC_ROWS)
    scv, scl = _sc_argmax(logits, _TC_ROWS)
    tail = logits[_TC_ROWS:, _TAIL0:].reshape(_NS, 8, V - _TAIL0)
    sc_ids = pl.pallas_call(
        _merge_kernel,
        in_specs=[
            pl.BlockSpec((_NS, 8, V - _TAIL0), lambda: (0, 0, 0)),
            pl.BlockSpec((_NW, _L), lambda: (0, 0)),
            pl.BlockSpec((_NW, _L), lambda: (0, 0)),
        ],
        out_specs=pl.BlockSpec((_NS, _L), lambda: (0, 0)),
        out_shape=jax.ShapeDtypeStruct((_NS, _L), jnp.int32),
    )(tail, scv, scl)
    out_ids = jnp.concatenate([tc_ids, sc_ids[:, :8].reshape(R - _TC_ROWS)])
    return _accept(out_ids.reshape(B, K + 1), spec_token_ids, B, K)


def _accept(out_ids, spec_token_ids, B, K):
    sampled, lengths = pl.pallas_call(
        _accept_kernel,
        in_specs=[
            pl.BlockSpec((B, K + 1), lambda: (0, 0)),
            pl.BlockSpec((B, K), lambda: (0, 0)),
        ],
        out_specs=[
            pl.BlockSpec((B, K + 1), lambda: (0, 0)),
            pl.BlockSpec((B, 1), lambda: (0, 0)),
        ],
        out_shape=[
            jax.ShapeDtypeStruct((B, K + 1), jnp.int32),
            jax.ShapeDtypeStruct((B, 1), jnp.int32),
        ],
    )(out_ids, spec_token_ids)
    return sampled, lengths.reshape(B)


# speedup vs baseline: 5.5349x; 1.0066x over previous
import functools

import jax
import jax.numpy as jnp
from jax.experimental import pallas as pl
from jax.experimental.pallas import tpu as pltpu


def _argmax_ring_kernel(x_hbm, out_ref, *bufs_sems, nbuf, rb, chunk):
    nrows, v = out_ref.shape[0], x_hbm.shape[1]
    bufs = bufs_sems[:nbuf]
    sems = bufs_sems[nbuf:]
    n = nrows // rb

    def copy(i):
        return pltpu.make_async_copy(
            x_hbm.at[pl.ds(i * rb, rb), :], bufs[i % nbuf], sems[i % nbuf])

    for i in range(min(nbuf, n)):
        copy(i).start()
    for i in range(n):
        copy(i).wait()
        buf = bufs[i % nbuf]
        run_max = jnp.full((rb,), -jnp.inf, dtype=jnp.float32)
        run_idx = jnp.zeros((rb,), dtype=jnp.int32)
        off = 0
        while off < v:
            w = min(chunk, v - off)
            xc = buf[:, off:off + w]
            cmax = jnp.max(xc, axis=1)
            cidx = jnp.argmax(xc, axis=1).astype(jnp.int32) + off
            upd = cmax > run_max
            run_max = jnp.where(upd, cmax, run_max)
            run_idx = jnp.where(upd, cidx, run_idx)
            off += w
        out_ref[pl.ds(i * rb, rb), :] = run_idx.reshape(rb, 1)
        if i + nbuf < n:
            copy(i + nbuf).start()


def _argmax_rows(logits, nbuf=8, rb=8, chunk=12800):
    nrows, v = logits.shape
    return pl.pallas_call(
        functools.partial(_argmax_ring_kernel, nbuf=nbuf, rb=rb, chunk=chunk),
        in_specs=[pl.BlockSpec(memory_space=pl.ANY)],
        out_specs=pl.BlockSpec((nrows, 1), lambda: (0, 0)),
        out_shape=jax.ShapeDtypeStruct((nrows, 1), jnp.int32),
        scratch_shapes=(
            [pltpu.VMEM((rb, v), jnp.float32)] * nbuf
            + [pltpu.SemaphoreType.DMA] * nbuf),
    )(logits)[:, 0]


def _accept_kernel(ids_ref, spec_ref, sampled_ref, len_ref):
    ids = ids_ref[...]
    spec = spec_ref[...]
    B, K1 = ids.shape
    K = K1 - 1
    prod = jnp.ones((B,), dtype=jnp.int32)
    total = jnp.zeros((B,), dtype=jnp.int32)
    for k in range(K):
        m = (ids[:, k] == spec[:, k]).astype(jnp.int32)
        prod = prod * m
        total = total + prod
    lengths = total + 1
    pos = jax.lax.broadcasted_iota(jnp.int32, (B, K1), 1)
    sampled_ref[...] = jnp.where(pos < lengths[:, None], ids, -1)
    len_ref[...] = lengths.reshape(B, 1)


def kernel(logits, spec_token_ids):
    B, K = spec_token_ids.shape
    out_ids = _argmax_rows(logits).reshape(B, K + 1)
    sampled, lengths = pl.pallas_call(
        _accept_kernel,
        in_specs=[
            pl.BlockSpec((B, K + 1), lambda: (0, 0)),
            pl.BlockSpec((B, K), lambda: (0, 0)),
        ],
        out_specs=[
            pl.BlockSpec((B, K + 1), lambda: (0, 0)),
            pl.BlockSpec((B, 1), lambda: (0, 0)),
        ],
        out_shape=[
            jax.ShapeDtypeStruct((B, K + 1), jnp.int32),
            jax.ShapeDtypeStruct((B, 1), jnp.int32),
        ],
    )(out_ids, spec_token_ids)
    return sampled, lengths.reshape(B)
